# baseline (speedup 1.0000x reference)
import functools

import jax
import jax.numpy as jnp
from jax import lax
from jax.experimental import pallas as pl
from jax.experimental.pallas import tpu as pltpu
from jax.experimental.pallas import tpu_sc as plsc

_GAMMA = 12.0

_NUM_CORES = 2
_NUM_SUBCORES = 16
_NW = _NUM_CORES * _NUM_SUBCORES
_BATCH = 16384
_D = 64
_DP = 128
_BPW = _BATCH // _NW
_CHUNK = 128
_NCHUNK = _BPW // _CHUNK


def _sc_body(hidx_hbm, ridx_hbm, tidx_hbm, ent_hbm, rel_hbm, out_hbm,
             hidx_v, ridx_v, tidx_v, hbuf, rbuf, tbuf, out_v, sem):
    wid = lax.axis_index("s") * _NUM_CORES + lax.axis_index("c")

    pltpu.sync_copy(hidx_hbm.at[wid], hidx_v)
    pltpu.sync_copy(ridx_hbm.at[wid], ridx_v)
    pltpu.sync_copy(tidx_hbm.at[wid], tidx_v)

    def fire(j):
        b = j % 2
        return [
            pltpu.async_copy(ent_hbm.at[hidx_v.at[j]], hbuf.at[b], sem),
            pltpu.async_copy(rel_hbm.at[ridx_v.at[j]], rbuf.at[b], sem),
            pltpu.async_copy(ent_hbm.at[tidx_v.at[j]], tbuf.at[b], sem),
        ]

    lanes = lax.iota(jnp.int32, 16)

    def compute(j):
        hb, rb, tb = hbuf.at[j % 2], rbuf.at[j % 2], tbuf.at[j % 2]

        def group(g, carry):
            rows = g * 16 + lanes
            acc = jnp.zeros((16,), jnp.float32)
            for d in range(_D):
                cols = jnp.full((16,), d, jnp.int32)
                h = plsc.load_gather(hb, [rows, cols])
                r = plsc.load_gather(rb, [rows, cols])
                t = plsc.load_gather(tb, [rows, cols])
                acc = acc + lax.abs(h + r - t)
            out_v[pl.ds(j * _CHUNK + g * 16, 16)] = _GAMMA - acc
            return carry

        lax.fori_loop(0, _CHUNK // 16, group, 0)

    inflight = fire(0)
    for j in range(_NCHUNK):
        for c in inflight:
            c.wait()
        if j + 1 < _NCHUNK:
            inflight = fire(j + 1)
        compute(j)

    pltpu.sync_copy(out_v, out_hbm.at[wid])


@jax.jit
def _transe_score(hidx, ridx, tidx, ent_p, rel_p):
    mesh = plsc.VectorSubcoreMesh(core_axis_name="c", subcore_axis_name="s")
    kfn = pl.kernel(
        _sc_body,
        out_type=jax.ShapeDtypeStruct((_NW, _BPW), jnp.float32),
        mesh=mesh,
        compiler_params=pltpu.CompilerParams(
            needs_layout_passes=False, use_tc_tiling_on_sc=True),
        scratch_types=[
            pltpu.VMEM((_NCHUNK, _CHUNK), jnp.int32),
            pltpu.VMEM((_NCHUNK, _CHUNK), jnp.int32),
            pltpu.VMEM((_NCHUNK, _CHUNK), jnp.int32),
            pltpu.VMEM((2, _CHUNK, _DP), jnp.float32),
            pltpu.VMEM((2, _CHUNK, _DP), jnp.float32),
            pltpu.VMEM((2, _CHUNK, _DP), jnp.float32),
            pltpu.VMEM((_BPW,), jnp.float32),
            pltpu.SemaphoreType.DMA,
        ],
    )
    return kfn(hidx, ridx, tidx, ent_p, rel_p)


def kernel(sample, ent_emb, relation_embedding):
    s = sample.astype(jnp.int32)
    hidx = s[:, 0].reshape(_NW, _NCHUNK, _CHUNK)
    ridx = s[:, 1].reshape(_NW, _NCHUNK, _CHUNK)
    tidx = s[:, 2].reshape(_NW, _NCHUNK, _CHUNK)
    ent_p = jnp.pad(ent_emb, ((0, 0), (0, _DP - _D)))
    rel_p = jnp.pad(relation_embedding, ((0, 0), (0, _DP - _D)))
    out = _transe_score(hidx, ridx, tidx, ent_p, rel_p)
    return out.reshape(_BATCH, 1)


# speedup vs baseline: 1.2824x; 1.2824x over previous
import functools

import jax
import jax.numpy as jnp
from jax import lax
from jax.experimental import pallas as pl
from jax.experimental.pallas import tpu as pltpu
from jax.experimental.pallas import tpu_sc as plsc

_GAMMA = 12.0

_NUM_CORES = 2
_NUM_SUBCORES = 16
_NW = _NUM_CORES * _NUM_SUBCORES
_BATCH = 16384
_D = 64
_DP = 128
_BPW = _BATCH // _NW
_CHUNK = 128
_NCHUNK = _BPW // _CHUNK


def _sc_body(hidx_hbm, ridx_hbm, tidx_hbm, ent_hbm, rel_hbm, out_hbm,
             hidx_v, ridx_v, tidx_v, hbuf, tbuf, out_v, sem):
    wid = lax.axis_index("s") * _NUM_CORES + lax.axis_index("c")

    pltpu.sync_copy(hidx_hbm.at[wid], hidx_v)
    pltpu.sync_copy(ridx_hbm.at[wid], ridx_v)
    pltpu.sync_copy(tidx_hbm.at[wid], tidx_v)

    def fire_ht(j):
        b = j % 2
        return [
            pltpu.async_copy(ent_hbm.at[hidx_v.at[j]], hbuf.at[b], sem),
            pltpu.async_copy(ent_hbm.at[tidx_v.at[j]], tbuf.at[b], sem),
        ]

    def fire_r(j):
        b = j % 2
        return [pltpu.async_copy(rel_hbm.at[ridx_v.at[j]], hbuf.at[b], sem,
                                 add=True)]

    last_lane = lax.iota(jnp.int32, 16) == 15

    def compute(j):
        b = j % 2

        def row(i, carry):
            acc = jnp.zeros((16,), jnp.float32)
            for c in range(_D // 16):
                sl = pl.ds(c * 16, 16)
                s = hbuf[b, i, sl] - tbuf[b, i, sl]
                acc = acc + lax.abs(s)
            tot = plsc.cumsum(acc)
            plsc.store_scatter(out_v, [jnp.full((16,), j * _CHUNK + i, jnp.int32)],
                               _GAMMA - tot, mask=last_lane)
            return carry

        lax.fori_loop(0, _CHUNK, row, 0, unroll=8)

    inflight_ht = fire_ht(0)
    inflight_r = []
    for j in range(_NCHUNK):
        for c in inflight_ht:
            c.wait()
        inflight_r = fire_r(j)
        if j + 1 < _NCHUNK:
            inflight_ht = fire_ht(j + 1)
        for c in inflight_r:
            c.wait()
        compute(j)

    pltpu.sync_copy(out_v, out_hbm.at[wid])


@jax.jit
def _transe_score(hidx, ridx, tidx, ent_p, rel_p):
    mesh = plsc.VectorSubcoreMesh(core_axis_name="c", subcore_axis_name="s")
    kfn = pl.kernel(
        _sc_body,
        out_type=jax.ShapeDtypeStruct((_NW, _BPW), jnp.float32),
        mesh=mesh,
        compiler_params=pltpu.CompilerParams(
            needs_layout_passes=False, use_tc_tiling_on_sc=True),
        scratch_types=[
            pltpu.VMEM((_NCHUNK, _CHUNK), jnp.int32),
            pltpu.VMEM((_NCHUNK, _CHUNK), jnp.int32),
            pltpu.VMEM((_NCHUNK, _CHUNK), jnp.int32),
            pltpu.VMEM((2, _CHUNK, _DP), jnp.float32),
            pltpu.VMEM((2, _CHUNK, _DP), jnp.float32),
            pltpu.VMEM((_BPW,), jnp.float32),
            pltpu.SemaphoreType.DMA,
        ],
    )
    return kfn(hidx, ridx, tidx, ent_p, rel_p)


def kernel(sample, ent_emb, relation_embedding):
    s = sample.astype(jnp.int32)
    hidx = s[:, 0].reshape(_NW, _NCHUNK, _CHUNK)
    ridx = s[:, 1].reshape(_NW, _NCHUNK, _CHUNK)
    tidx = s[:, 2].reshape(_NW, _NCHUNK, _CHUNK)
    ent_p = jnp.pad(ent_emb, ((0, 0), (0, _DP - _D)))
    rel_p = jnp.pad(relation_embedding, ((0, 0), (0, _DP - _D)))
    out = _transe_score(hidx, ridx, tidx, ent_p, rel_p)
    return out.reshape(_BATCH, 1)


# speedup vs baseline: 1.3200x; 1.0293x over previous
import functools

import jax
import jax.numpy as jnp
from jax import lax
from jax.experimental import pallas as pl
from jax.experimental.pallas import tpu as pltpu
from jax.experimental.pallas import tpu_sc as plsc

_GAMMA = 12.0

_NUM_CORES = 2
_NUM_SUBCORES = 16
_NW = _NUM_CORES * _NUM_SUBCORES
_BATCH = 16384
_D = 64
_DP = 128
_BPW = _BATCH // _NW
_CHUNK = 128
_NCHUNK = _BPW // _CHUNK


def _sc_body(hidx_hbm, ridx_hbm, tidx_hbm, ent_hbm, rel_hbm, out_hbm,
             hidx_v, ridx_v, tidx_v, hbuf, rbuf, tbuf, out_v, sem):
    wid = lax.axis_index("s") * _NUM_CORES + lax.axis_index("c")

    pltpu.sync_copy(hidx_hbm.at[wid], hidx_v)
    pltpu.sync_copy(ridx_hbm.at[wid], ridx_v)
    pltpu.sync_copy(tidx_hbm.at[wid], tidx_v)

    def fire(j):
        b = j % 2
        return [
            pltpu.async_copy(ent_hbm.at[hidx_v.at[j]], hbuf.at[b], sem),
            pltpu.async_copy(rel_hbm.at[ridx_v.at[j]], rbuf.at[b], sem),
            pltpu.async_copy(ent_hbm.at[tidx_v.at[j]], tbuf.at[b], sem),
        ]

    last_lane = lax.iota(jnp.int32, 16) == 15

    def compute(j):
        b = j % 2

        def row(i, carry):
            acc = jnp.zeros((16,), jnp.float32)
            for c in range(_D // 16):
                sl = pl.ds(c * 16, 16)
                s = hbuf[b, i, sl] + rbuf[b, i, sl] - tbuf[b, i, sl]
                acc = acc + lax.abs(s)
            tot = plsc.cumsum(acc)
            plsc.store_scatter(out_v, [jnp.full((16,), j * _CHUNK + i, jnp.int32)],
                               _GAMMA - tot, mask=last_lane)
            return carry

        lax.fori_loop(0, _CHUNK, row, 0, unroll=8)

    inflight = fire(0)
    for j in range(_NCHUNK):
        for c in inflight:
            c.wait()
        if j + 1 < _NCHUNK:
            inflight = fire(j + 1)
        compute(j)

    pltpu.sync_copy(out_v, out_hbm.at[wid])


@jax.jit
def _transe_score(hidx, ridx, tidx, ent_p, rel_p):
    mesh = plsc.VectorSubcoreMesh(core_axis_name="c", subcore_axis_name="s")
    kfn = pl.kernel(
        _sc_body,
        out_type=jax.ShapeDtypeStruct((_NW, _BPW), jnp.float32),
        mesh=mesh,
        compiler_params=pltpu.CompilerParams(
            needs_layout_passes=False, use_tc_tiling_on_sc=True),
        scratch_types=[
            pltpu.VMEM((_NCHUNK, _CHUNK), jnp.int32),
            pltpu.VMEM((_NCHUNK, _CHUNK), jnp.int32),
            pltpu.VMEM((_NCHUNK, _CHUNK), jnp.int32),
            pltpu.VMEM((2, _CHUNK, _DP), jnp.float32),
            pltpu.VMEM((2, _CHUNK, _DP), jnp.float32),
            pltpu.VMEM((2, _CHUNK, _DP), jnp.float32),
            pltpu.VMEM((_BPW,), jnp.float32),
            pltpu.SemaphoreType.DMA,
        ],
    )
    return kfn(hidx, ridx, tidx, ent_p, rel_p)


def kernel(sample, ent_emb, relation_embedding):
    s = sample.astype(jnp.int32)
    hidx = s[:, 0].reshape(_NW, _NCHUNK, _CHUNK)
    ridx = s[:, 1].reshape(_NW, _NCHUNK, _CHUNK)
    tidx = s[:, 2].reshape(_NW, _NCHUNK, _CHUNK)
    ent_p = jnp.pad(ent_emb, ((0, 0), (0, _DP - _D)))
    rel_p = jnp.pad(relation_embedding, ((0, 0), (0, _DP - _D)))
    out = _transe_score(hidx, ridx, tidx, ent_p, rel_p)
    return out.reshape(_BATCH, 1)
